# Initial kernel scaffold; baseline (speedup 1.0000x reference)
import jax
import jax.numpy as jnp
from jax.experimental import pallas as pl


def kernel(x, index, input):
    raise NotImplementedError("write your pallas kernel here")


# speedup vs baseline: 21.9078x; 21.9078x over previous
import jax
import jax.numpy as jnp
from jax import lax
from jax.experimental import pallas as pl
from jax.experimental.pallas import tpu as pltpu
from jax.experimental.pallas import tpu_sc as plsc

NSRC = 320000
NOUT = 10000
D = 128
NC = 2
NS = 16
LANES = 16
NG = D // LANES
GPC = NG // NC
RS = 4
ROWS = NOUT // RS
ACC = ROWS * LANES
HROWS = NOUT // 2
HACC = HROWS * LANES
CH = 2048
NCHUNK = NSRC * LANES // CH
GSTRIDE = NSRC * LANES
OSTRIDE = NOUT * LANES

_params = pltpu.CompilerParams(needs_layout_passes=False)


def _dma_in(xT, iT, off, xb, ib, sx, si):
    pltpu.async_copy(xT.at[pl.ds(off, CH)], xb, sx)
    pltpu.async_copy(iT.at[pl.ds(off, CH)], ib, si)


def _dma_wait(xT, iT, off, xb, ib, sx, si):
    pltpu.make_async_copy(xT.at[pl.ds(off, CH)], xb, sx).wait()
    pltpu.make_async_copy(iT.at[pl.ds(off, CH)], ib, si).wait()


def _rmw_body(xT, iT, inpT, omax, omin, omul,
              accmax, accmin, accmul, x0, x1, i0, i1, sx0, si0, sx1, si1):
    c = lax.axis_index("c")
    s = lax.axis_index("s")
    g = c * GPC + s // RS
    r = s % RS
    row0 = r * ROWS
    gbase = g * GSTRIDE
    obase = g * OSTRIDE + row0 * LANES
    lane = lax.iota(jnp.int32, LANES)

    pltpu.sync_copy(inpT.at[pl.ds(obase, ACC)], accmax)
    pltpu.sync_copy(inpT.at[pl.ds(obase, ACC)], accmin)
    pltpu.sync_copy(inpT.at[pl.ds(obase, ACC)], accmul)

    def process(xb, ib):
        def inner(i, icarry):
            ix = ib[pl.ds(i * LANES, LANES)]
            v = xb[pl.ds(i * LANES, LANES)]
            t = ix - row0
            m = (t >= 0) & (t < ROWS)
            tcl = lax.min(lax.max(t, 0), ROWS - 1)
            addr = tcl * LANES + lane
            cm = plsc.load_gather(accmax, [addr], mask=m)
            plsc.store_scatter(accmax, [addr], jnp.maximum(cm, v), mask=m)
            cn = plsc.load_gather(accmin, [addr], mask=m)
            plsc.store_scatter(accmin, [addr], jnp.minimum(cn, v), mask=m)
            cp = plsc.load_gather(accmul, [addr], mask=m)
            plsc.store_scatter(accmul, [addr], cp * v, mask=m)
            return icarry
        lax.fori_loop(0, CH // LANES, inner, 0)

    _dma_in(xT, iT, gbase, x0, i0, sx0, si0)

    def pair(k, carry):
        c0 = 2 * k
        c1 = 2 * k + 1
        _dma_in(xT, iT, gbase + c1 * CH, x1, i1, sx1, si1)
        _dma_wait(xT, iT, gbase + c0 * CH, x0, i0, sx0, si0)
        process(x0, i0)

        @pl.when(k < NCHUNK // 2 - 1)
        def _():
            _dma_in(xT, iT, gbase + (c1 + 1) * CH, x0, i0, sx0, si0)
        _dma_wait(xT, iT, gbase + c1 * CH, x1, i1, sx1, si1)
        process(x1, i1)
        return carry

    lax.fori_loop(0, NCHUNK // 2, pair, 0)

    pltpu.sync_copy(accmax, omax.at[pl.ds(obase, ACC)])
    pltpu.sync_copy(accmin, omin.at[pl.ds(obase, ACC)])
    pltpu.sync_copy(accmul, omul.at[pl.ds(obase, ACC)])


def _sum_body(xT, iT, osum, accsum, x0, x1, i0, i1, sx0, si0, sx1, si1):
    c = lax.axis_index("c")
    s = lax.axis_index("s")
    g = c * GPC + s // RS
    rh = (s // 2) % 2
    sh = s % 2
    row0 = rh * HROWS
    gbase = g * GSTRIDE
    lane = lax.iota(jnp.int32, LANES)

    def zfill(i, carry):
        accsum[pl.ds(i * LANES, LANES)] = jnp.zeros((LANES,), jnp.float32)
        return carry
    lax.fori_loop(0, HACC // LANES, zfill, 0)

    def process(xb, ib):
        def inner(i, icarry):
            ix = ib[pl.ds(i * LANES, LANES)]
            v = xb[pl.ds(i * LANES, LANES)]
            t = ix - row0
            m = (t >= 0) & (t < HROWS)
            tcl = lax.min(lax.max(t, 0), HROWS - 1)
            addr = tcl * LANES + lane
            plsc.addupdate_scatter(accsum, [addr], v, mask=m)
            return icarry
        lax.fori_loop(0, CH // LANES, inner, 0)

    nh = NCHUNK // 2
    cbase = sh * nh
    _dma_in(xT, iT, gbase + cbase * CH, x0, i0, sx0, si0)

    def pair(k, carry):
        c0 = cbase + 2 * k
        c1 = cbase + 2 * k + 1
        _dma_in(xT, iT, gbase + c1 * CH, x1, i1, sx1, si1)
        _dma_wait(xT, iT, gbase + c0 * CH, x0, i0, sx0, si0)
        process(x0, i0)

        @pl.when(k < nh // 2 - 1)
        def _():
            _dma_in(xT, iT, gbase + (c1 + 1) * CH, x0, i0, sx0, si0)
        _dma_wait(xT, iT, gbase + c1 * CH, x1, i1, sx1, si1)
        process(x1, i1)
        return carry

    lax.fori_loop(0, nh // 2, pair, 0)

    obase = sh * (NG * OSTRIDE) + g * OSTRIDE + row0 * LANES
    pltpu.sync_copy(accsum, osum.at[pl.ds(obase, HACC)])


def _sc_rmw(xT, iT, inpT):
    mesh = plsc.VectorSubcoreMesh(core_axis_name="c", subcore_axis_name="s")
    f = pl.kernel(
        _rmw_body,
        mesh=mesh,
        compiler_params=_params,
        out_type=[
            jax.ShapeDtypeStruct((NG * OSTRIDE,), jnp.float32),
            jax.ShapeDtypeStruct((NG * OSTRIDE,), jnp.float32),
            jax.ShapeDtypeStruct((NG * OSTRIDE,), jnp.float32),
        ],
        scratch_types=[
            pltpu.VMEM((ACC,), jnp.float32),
            pltpu.VMEM((ACC,), jnp.float32),
            pltpu.VMEM((ACC,), jnp.float32),
            pltpu.VMEM((CH,), jnp.float32),
            pltpu.VMEM((CH,), jnp.float32),
            pltpu.VMEM((CH,), jnp.int32),
            pltpu.VMEM((CH,), jnp.int32),
            pltpu.SemaphoreType.DMA,
            pltpu.SemaphoreType.DMA,
            pltpu.SemaphoreType.DMA,
            pltpu.SemaphoreType.DMA,
        ],
    )
    return f(xT, iT, inpT)


def _sc_sum(xT, iT):
    mesh = plsc.VectorSubcoreMesh(core_axis_name="c", subcore_axis_name="s")
    f = pl.kernel(
        _sum_body,
        mesh=mesh,
        compiler_params=_params,
        out_type=jax.ShapeDtypeStruct((2 * NG * OSTRIDE,), jnp.float32),
        scratch_types=[
            pltpu.VMEM((HACC,), jnp.float32),
            pltpu.VMEM((CH,), jnp.float32),
            pltpu.VMEM((CH,), jnp.float32),
            pltpu.VMEM((CH,), jnp.int32),
            pltpu.VMEM((CH,), jnp.int32),
            pltpu.SemaphoreType.DMA,
            pltpu.SemaphoreType.DMA,
            pltpu.SemaphoreType.DMA,
            pltpu.SemaphoreType.DMA,
        ],
    )
    return f(xT, iT)


def kernel(x, index, input):
    xT = x.reshape(NSRC, NG, LANES).transpose(1, 0, 2).reshape(-1)
    iT = index.reshape(NSRC, NG, LANES).transpose(1, 0, 2).reshape(-1)
    inpT = input.reshape(NOUT, NG, LANES).transpose(1, 0, 2).reshape(-1)
    omax, omin, omul = _sc_rmw(xT, iT, inpT)
    osum = _sc_sum(xT, iT)

    def back(o):
        return o.reshape(NG, NOUT, LANES).transpose(1, 0, 2).reshape(NOUT, D)

    y_max = back(omax)
    y_min = back(omin)
    y_mul = back(omul)
    op = osum.reshape(2, NG, NOUT, LANES)
    y_sum = ((op[0] + op[1]).transpose(1, 0, 2).reshape(NOUT, D) + input)
    return (y_max, y_sum, y_min, y_mul)


# speedup vs baseline: 49.8154x; 2.2739x over previous
import jax
import jax.numpy as jnp
from jax import lax
from jax.experimental import pallas as pl
from jax.experimental.pallas import tpu as pltpu
from jax.experimental.pallas import tpu_sc as plsc

NSRC = 320000
NOUT = 10000
D = 128
NC = 2
NS = 16
LANES = 16
NG = D // LANES
GPC = NG // NC
RS = 4
ROWS = NOUT // RS
ACC = ROWS * LANES
HROWS = NOUT // 2
HACC = HROWS * LANES
CH = 2048
NCHUNK = NSRC * LANES // CH
GSTRIDE = NSRC * LANES
OSTRIDE = NOUT * LANES

_params = pltpu.CompilerParams(needs_layout_passes=False)


def _dma_in(xT, iT, off, xb, ib, sx, si):
    pltpu.async_copy(xT.at[pl.ds(off, CH)], xb, sx)
    pltpu.async_copy(iT.at[pl.ds(off, CH)], ib, si)


def _dma_wait(xT, iT, off, xb, ib, sx, si):
    pltpu.make_async_copy(xT.at[pl.ds(off, CH)], xb, sx).wait()
    pltpu.make_async_copy(iT.at[pl.ds(off, CH)], ib, si).wait()


def _rmw_body(xT, iT, inpT, omax, omin, omul,
              accmax, accmin, accmul, x0, x1, i0, i1, sx0, si0, sx1, si1):
    c = lax.axis_index("c")
    s = lax.axis_index("s")
    g = c * GPC + s // RS
    r = s % RS
    row0 = r * ROWS
    gbase = g * GSTRIDE
    obase = g * OSTRIDE + row0 * LANES
    lane = lax.iota(jnp.int32, LANES)

    pltpu.sync_copy(inpT.at[pl.ds(obase, ACC)], accmax.at[pl.ds(0, ACC)])
    pltpu.sync_copy(inpT.at[pl.ds(obase, ACC)], accmin.at[pl.ds(0, ACC)])
    pltpu.sync_copy(inpT.at[pl.ds(obase, ACC)], accmul.at[pl.ds(0, ACC)])

    laneadj = lane - row0 * LANES
    dumpvec = jnp.int32(ACC) + lane

    def addr_of(xb, ib, k):
        ix = ib[pl.ds(k * LANES, LANES)]
        v = xb[pl.ds(k * LANES, LANES)]
        a = (ix << 4) + laneadj
        addr = jnp.minimum(a.astype(jnp.uint32),
                           dumpvec.astype(jnp.uint32)).astype(jnp.int32)
        return v, addr

    def process(xb, ib):
        def chains(v, addr):
            cm = plsc.load_gather(accmax, [addr])
            cn = plsc.load_gather(accmin, [addr])
            cp = plsc.load_gather(accmul, [addr])
            plsc.store_scatter(accmax, [addr], jnp.maximum(cm, v))
            plsc.store_scatter(accmin, [addr], jnp.minimum(cn, v))
            plsc.store_scatter(accmul, [addr], cp * v)

        n = CH // LANES

        def inner(i, carry):
            e0, e1 = carry
            f0 = addr_of(xb, ib, 2 * i + 2)
            f1 = addr_of(xb, ib, 2 * i + 3)
            chains(*e0)
            chains(*e1)
            return (f0, f1)
        e0, e1 = lax.fori_loop(
            0, (n - 2) // 2, inner,
            (addr_of(xb, ib, 0), addr_of(xb, ib, 1)))
        chains(*e0)
        chains(*e1)

    _dma_in(xT, iT, gbase, x0, i0, sx0, si0)

    def pair(k, carry):
        c0 = 2 * k
        c1 = 2 * k + 1
        _dma_in(xT, iT, gbase + c1 * CH, x1, i1, sx1, si1)
        _dma_wait(xT, iT, gbase + c0 * CH, x0, i0, sx0, si0)
        process(x0, i0)

        @pl.when(k < NCHUNK // 2 - 1)
        def _():
            _dma_in(xT, iT, gbase + (c1 + 1) * CH, x0, i0, sx0, si0)
        _dma_wait(xT, iT, gbase + c1 * CH, x1, i1, sx1, si1)
        process(x1, i1)
        return carry

    lax.fori_loop(0, NCHUNK // 2, pair, 0)

    pltpu.sync_copy(accmax.at[pl.ds(0, ACC)], omax.at[pl.ds(obase, ACC)])
    pltpu.sync_copy(accmin.at[pl.ds(0, ACC)], omin.at[pl.ds(obase, ACC)])
    pltpu.sync_copy(accmul.at[pl.ds(0, ACC)], omul.at[pl.ds(obase, ACC)])


def _sum_body(xT, iT, osum, accsum, x0, x1, i0, i1, sx0, si0, sx1, si1):
    c = lax.axis_index("c")
    s = lax.axis_index("s")
    g = c * GPC + s // RS
    rh = (s // 2) % 2
    sh = s % 2
    row0 = rh * HROWS
    gbase = g * GSTRIDE
    lane = lax.iota(jnp.int32, LANES)

    def zfill(i, carry):
        accsum[pl.ds(i * LANES, LANES)] = jnp.zeros((LANES,), jnp.float32)
        return carry
    lax.fori_loop(0, HACC // LANES + 1, zfill, 0)

    laneadj = lane - row0 * LANES
    dumpvec = jnp.int32(HACC) + lane

    def addr_of(xb, ib, k):
        ix = ib[pl.ds(k * LANES, LANES)]
        v = xb[pl.ds(k * LANES, LANES)]
        a = (ix << 4) + laneadj
        addr = jnp.minimum(a.astype(jnp.uint32),
                           dumpvec.astype(jnp.uint32)).astype(jnp.int32)
        return v, addr

    def process(xb, ib):
        n = CH // LANES

        def inner(i, carry):
            e0, e1 = carry
            f0 = addr_of(xb, ib, 2 * i + 2)
            f1 = addr_of(xb, ib, 2 * i + 3)
            plsc.addupdate_scatter(accsum, [e0[1]], e0[0])
            plsc.addupdate_scatter(accsum, [e1[1]], e1[0])
            return (f0, f1)
        e0, e1 = lax.fori_loop(
            0, (n - 2) // 2, inner, (addr_of(xb, ib, 0), addr_of(xb, ib, 1)))
        for v, addr in (e0, e1):
            plsc.addupdate_scatter(accsum, [addr], v)

    nh = NCHUNK // 2
    cbase = sh * nh
    _dma_in(xT, iT, gbase + cbase * CH, x0, i0, sx0, si0)

    def pair(k, carry):
        c0 = cbase + 2 * k
        c1 = cbase + 2 * k + 1
        _dma_in(xT, iT, gbase + c1 * CH, x1, i1, sx1, si1)
        _dma_wait(xT, iT, gbase + c0 * CH, x0, i0, sx0, si0)
        process(x0, i0)

        @pl.when(k < nh // 2 - 1)
        def _():
            _dma_in(xT, iT, gbase + (c1 + 1) * CH, x0, i0, sx0, si0)
        _dma_wait(xT, iT, gbase + c1 * CH, x1, i1, sx1, si1)
        process(x1, i1)
        return carry

    lax.fori_loop(0, nh // 2, pair, 0)

    obase = sh * (NG * OSTRIDE) + g * OSTRIDE + row0 * LANES
    pltpu.sync_copy(accsum.at[pl.ds(0, HACC)], osum.at[pl.ds(obase, HACC)])


def _sc_rmw(xT, iT, inpT):
    mesh = plsc.VectorSubcoreMesh(core_axis_name="c", subcore_axis_name="s")
    f = pl.kernel(
        _rmw_body,
        mesh=mesh,
        compiler_params=_params,
        out_type=[
            jax.ShapeDtypeStruct((NG * OSTRIDE,), jnp.float32),
            jax.ShapeDtypeStruct((NG * OSTRIDE,), jnp.float32),
            jax.ShapeDtypeStruct((NG * OSTRIDE,), jnp.float32),
        ],
        scratch_types=[
            pltpu.VMEM((ACC + LANES,), jnp.float32),
            pltpu.VMEM((ACC + LANES,), jnp.float32),
            pltpu.VMEM((ACC + LANES,), jnp.float32),
            pltpu.VMEM((CH,), jnp.float32),
            pltpu.VMEM((CH,), jnp.float32),
            pltpu.VMEM((CH,), jnp.int32),
            pltpu.VMEM((CH,), jnp.int32),
            pltpu.SemaphoreType.DMA,
            pltpu.SemaphoreType.DMA,
            pltpu.SemaphoreType.DMA,
            pltpu.SemaphoreType.DMA,
        ],
    )
    return f(xT, iT, inpT)


def _sc_sum(xT, iT):
    mesh = plsc.VectorSubcoreMesh(core_axis_name="c", subcore_axis_name="s")
    f = pl.kernel(
        _sum_body,
        mesh=mesh,
        compiler_params=_params,
        out_type=jax.ShapeDtypeStruct((2 * NG * OSTRIDE,), jnp.float32),
        scratch_types=[
            pltpu.VMEM((HACC + LANES,), jnp.float32),
            pltpu.VMEM((CH,), jnp.float32),
            pltpu.VMEM((CH,), jnp.float32),
            pltpu.VMEM((CH,), jnp.int32),
            pltpu.VMEM((CH,), jnp.int32),
            pltpu.SemaphoreType.DMA,
            pltpu.SemaphoreType.DMA,
            pltpu.SemaphoreType.DMA,
            pltpu.SemaphoreType.DMA,
        ],
    )
    return f(xT, iT)


def kernel(x, index, input):
    xT = x.reshape(NSRC, NG, LANES).transpose(1, 0, 2).reshape(-1)
    iT = index.reshape(NSRC, NG, LANES).transpose(1, 0, 2).reshape(-1)
    inpT = input.reshape(NOUT, NG, LANES).transpose(1, 0, 2).reshape(-1)
    omax, omin, omul = _sc_rmw(xT, iT, inpT)
    osum = _sc_sum(xT, iT)

    def back(o):
        return o.reshape(NG, NOUT, LANES).transpose(1, 0, 2).reshape(NOUT, D)

    y_max = back(omax)
    y_min = back(omin)
    y_mul = back(omul)
    op = osum.reshape(2, NG, NOUT, LANES)
    y_sum = ((op[0] + op[1]).transpose(1, 0, 2).reshape(NOUT, D) + input)
    return (y_max, y_sum, y_min, y_mul)
